# baseline (speedup 1.0000x reference)
import jax
import jax.numpy as jnp
import numpy as np
from jax import lax
from jax.experimental import pallas as pl
from jax.experimental.pallas import tpu as pltpu
from jax.experimental.pallas import tpu_sc as plsc

_PROB = 0.25
_MAX_OFFSET = 256

_NC = 2
_NS = 16
_NW = _NC * _NS
_SR = 4


def _glitch_meta(n_rows, L, T, B):
    key = jax.random.key(42)
    kmask, key = jax.random.split(key)
    masks = jax.random.uniform(kmask, (2, B)) < _PROB
    center = L // 2
    min_start = center - T // 2 - _MAX_OFFSET
    max_start = center - T // 2 + _MAX_OFFSET
    idxs, starts = [], []
    for _ in range(2):
        kidx, kker, key = jax.random.split(key, 3)
        idxs.append(jax.random.randint(kidx, (B,), 0, n_rows))
        starts.append(jax.random.randint(kker, (B,), min_start, max_start + 1))
    masks = masks.astype(jnp.int32)
    idx = jnp.stack(idxs).astype(jnp.int32)
    st = jnp.stack(starts).astype(jnp.int32)

    nstep = B // _SR
    st_m = jnp.where(masks == 1, st, -1)

    def steps(a):
        a3 = a.reshape(2, nstep, _SR)
        return jnp.concatenate([a3[0], a3[1]], axis=1)

    sa_steps = steps(st_m)
    try:
        cnt = np.asarray((sa_steps >= 0).sum(axis=1))
        order_by_load = np.argsort(-cnt, kind="stable")
        tot = np.zeros(_NW, np.int64)
        fill = np.zeros(_NW, np.int64)
        spw = nstep // _NW
        assign = [[] for _ in range(_NW)]
        for t in order_by_load:
            cands = np.where(fill < spw)[0]
            w = cands[np.argmin(tot[cands])]
            assign[w].append(int(t))
            tot[w] += cnt[t]
            fill[w] += 1
        order = np.concatenate([np.asarray(a, np.int64) for a in assign])
    except jax.errors.TracerArrayConversionError:
        order = np.arange(nstep, dtype=np.int64)

    pad = jnp.zeros((nstep, 7), jnp.int32)
    row0 = (jnp.asarray(order, jnp.int32) * _SR)[:, None]
    sa = jnp.concatenate([sa_steps[order], row0, pad], axis=1).reshape(-1)

    i_step = steps(idx)
    ia8 = jnp.zeros((nstep, 2 * _SR, 8), jnp.int32).at[:, :, 0].set(i_step)
    ia8 = ia8[order]

    masks_flat = masks.reshape(-1)

    def mk(a):
        a3 = a.reshape(2, nstep, _SR)
        padz = jnp.zeros((nstep, 8), a.dtype)
        return jnp.concatenate([a3[0], a3[1], padz], axis=1).reshape(-1)

    return (masks_flat, ia8.reshape(-1), sa)


def _pick(vec, l):
    return vec[l]


def _sc_body(X, y, H1, L1, mflat, ia8, sa, Xo, yo,
             ia8_v, sa_v, m0_v, m1_v, yin_v, yout_v, slabA, slabB,
             b0, b1, b2, b3, b4, b5, b6, b7,
             sem_meta, sem_si, sem_so, sem_g):
    B, C, T = X.shape
    RPW = B // _NW
    NSL = RPW // _SR
    wid = lax.axis_index("s") * _NC + lax.axis_index("c")
    base = pl.multiple_of(wid * RPW, RPW)
    t0e = pl.multiple_of(wid * (NSL * 16), NSL * 16)
    t08 = pl.multiple_of(wid * (NSL * 64), NSL * 64)

    pltpu.async_copy(ia8.at[pl.ds(t08, NSL * 64)], ia8_v, sem_meta)
    pltpu.async_copy(sa.at[pl.ds(t0e, NSL * 16)], sa_v, sem_meta)
    pltpu.async_copy(mflat.at[pl.ds(base, RPW)], m0_v, sem_meta)
    pltpu.async_copy(mflat.at[pl.ds(B + base, RPW)], m1_v, sem_meta)
    pltpu.async_copy(y.at[pl.ds(base, RPW)], yin_v, sem_meta)
    pltpu.make_async_copy(sa.at[pl.ds(t0e, NSL * 16)], sa_v, sem_meta).wait()
    pltpu.make_async_copy(ia8.at[pl.ds(t08, NSL * 64)], ia8_v, sem_meta).wait()
    for ref in (m0_v, m1_v):
        pltpu.make_async_copy(mflat.at[pl.ds(base, RPW)], ref, sem_meta).wait()
    pltpu.make_async_copy(y.at[pl.ds(base, RPW)], yin_v, sem_meta).wait()

    for t in range(RPW // 16):
        sl = pl.ds(t * 16, 16)
        m0f = m0_v[sl].astype(jnp.float32)
        m1f = m1_v[sl].astype(jnp.float32)
        yout_v[sl] = yin_v[sl] - 2.0 * m0f - 4.0 * m1f
    pltpu.async_copy(yout_v, yo.at[pl.ds(base, RPW)], sem_meta)

    tabs = (H1, L1)
    banks = (b0, b1, b2, b3, b4, b5, b6, b7)

    def pick_step(s):
        moff = pl.multiple_of(s * 16, 16)
        sv = sa_v[pl.ds(moff, 16)]
        return tuple(_pick(sv, lanei) for lanei in range(2 * _SR + 1))

    def fire_gathers(s, sts):
        for lanei in range(2 * _SR):
            ch = lanei // _SR
            ioff = pl.multiple_of((s * (2 * _SR) + lanei) * 8, 8)

            @pl.when(sts[lanei] >= 0)
            def _():
                pltpu.async_copy(tabs[ch].at[ia8_v.at[pl.ds(ioff, 1)]],
                                 banks[lanei], sem_g)

    def fire_slab_in(sts, slab):
        for lanei in range(2 * _SR):
            ch = lanei // _SR
            l = lanei % _SR

            @pl.when(sts[lanei] < 0)
            def _():
                pltpu.async_copy(X.at[sts[2 * _SR] + l, ch],
                                 slab.at[l, ch], sem_si)

    sts0 = pick_step(0)
    fire_slab_in(sts0, slabA)
    fire_gathers(0, sts0)

    def process_step(s, sts, slab, other):
        row0 = sts[2 * _SR]
        nxt = pick_step(jnp.minimum(s + 1, NSL - 1))

        @pl.when(s >= 1)
        def _():
            pltpu.make_async_copy(other, Xo.at[pl.ds(base, _SR)],
                                  sem_so).wait()

        @pl.when(s < NSL - 1)
        def _():
            fire_slab_in(nxt, other)

        for lanei in range(2 * _SR):
            ch = lanei // _SR
            l = lanei % _SR

            @pl.when(sts[lanei] < 0)
            def _():
                pltpu.make_async_copy(X.at[0, ch], slab.at[l, ch],
                                      sem_si).wait()
        for lanei in range(2 * _SR):
            ch = lanei // _SR

            @pl.when(sts[lanei] >= 0)
            def _():
                pltpu.make_async_copy(tabs[ch].at[ia8_v.at[pl.ds(0, 1)]],
                                      banks[lanei], sem_g).wait()

        for lanei in range(2 * _SR):
            ch = lanei // _SR
            l = lanei % _SR
            st_s = sts[lanei]

            @pl.when(st_s >= 0)
            def _():
                def rep(k, _):
                    off = k * 64
                    for u in range(4):
                        slab[l, ch, pl.ds(off + u * 16, 16)] = (
                            banks[lanei][0, pl.ds(st_s + off + u * 16, 16)])
                    return _

                lax.fori_loop(0, T // 64, rep, None)

        @pl.when(s < NSL - 1)
        def _():
            fire_gathers(s + 1, nxt)

        pltpu.async_copy(slab, Xo.at[pl.ds(row0, _SR)], sem_so)
        return nxt

    def step2(s2, sts):
        sts = process_step(2 * s2, sts, slabA, slabB)
        sts = process_step(2 * s2 + 1, sts, slabB, slabA)
        return sts

    lax.fori_loop(0, NSL // 2, step2, sts0)
    pltpu.make_async_copy(slabB, Xo.at[pl.ds(base, _SR)], sem_so).wait()
    pltpu.make_async_copy(yout_v, yo.at[pl.ds(base, RPW)], sem_meta).wait()


def kernel(X, y, H1, L1):
    B, C, T = X.shape
    try:
        with jax.ensure_compile_time_eval(), \
                jax.default_device(jax.local_devices(backend="cpu")[0]):
            meta = _glitch_meta(H1.shape[0], H1.shape[1], T, B)
        mflat, ia, sa = (jnp.asarray(np.asarray(a)) for a in meta)
    except Exception:
        mflat, ia, sa = _glitch_meta(H1.shape[0], H1.shape[1], T, B)
    RPW = B // _NW
    NSL = RPW // _SR
    mesh = plsc.VectorSubcoreMesh(core_axis_name="c", subcore_axis_name="s",
                                  num_cores=_NC, num_subcores=_NS)
    run = pl.kernel(
        _sc_body,
        out_type=(jax.ShapeDtypeStruct((B, C, T), jnp.float32),
                  jax.ShapeDtypeStruct((B,), jnp.float32)),
        mesh=mesh,
        scratch_types=[
            pltpu.VMEM((NSL * 64,), jnp.int32),
            pltpu.VMEM((NSL * 16,), jnp.int32),
            pltpu.VMEM((RPW,), jnp.int32),
            pltpu.VMEM((RPW,), jnp.int32),
            pltpu.VMEM((RPW,), jnp.float32),
            pltpu.VMEM((RPW,), jnp.float32),
            pltpu.VMEM((_SR, C, T), jnp.float32),
            pltpu.VMEM((_SR, C, T), jnp.float32),
            pltpu.VMEM((1, 4096), jnp.float32),
            pltpu.VMEM((1, 4096), jnp.float32),
            pltpu.VMEM((1, 4096), jnp.float32),
            pltpu.VMEM((1, 4096), jnp.float32),
            pltpu.VMEM((1, 4096), jnp.float32),
            pltpu.VMEM((1, 4096), jnp.float32),
            pltpu.VMEM((1, 4096), jnp.float32),
            pltpu.VMEM((1, 4096), jnp.float32),
            pltpu.SemaphoreType.DMA,
            pltpu.SemaphoreType.DMA,
            pltpu.SemaphoreType.DMA,
            pltpu.SemaphoreType.DMA,
        ],
        compiler_params=pltpu.CompilerParams(needs_layout_passes=False),
    )
    Xo, yo = run(X, y, H1, L1, mflat, ia, sa)
    return (Xo, yo)


# speedup vs baseline: 1.0900x; 1.0900x over previous
import jax
import jax.numpy as jnp
import numpy as np
from jax import lax
from jax.experimental import pallas as pl
from jax.experimental.pallas import tpu as pltpu
from jax.experimental.pallas import tpu_sc as plsc

_PROB = 0.25
_MAX_OFFSET = 256

_NC = 2
_NS = 16
_NW = _NC * _NS
_SR = 4


def _glitch_meta(n_rows, L, T, B):
    key = jax.random.key(42)
    kmask, key = jax.random.split(key)
    masks = jax.random.uniform(kmask, (2, B)) < _PROB
    center = L // 2
    min_start = center - T // 2 - _MAX_OFFSET
    max_start = center - T // 2 + _MAX_OFFSET
    idxs, starts = [], []
    for _ in range(2):
        kidx, kker, key = jax.random.split(key, 3)
        idxs.append(jax.random.randint(kidx, (B,), 0, n_rows))
        starts.append(jax.random.randint(kker, (B,), min_start, max_start + 1))
    masks = masks.astype(jnp.int32)
    idx = jnp.stack(idxs).astype(jnp.int32)
    st = jnp.stack(starts).astype(jnp.int32)

    nstep = B // _SR
    st_m = jnp.where(masks == 1, st, -1)

    def steps(a):
        a3 = a.reshape(2, nstep, _SR)
        return jnp.concatenate([a3[0], a3[1]], axis=1)

    sa_steps = steps(st_m)
    try:
        cnt = np.asarray((sa_steps >= 0).sum(axis=1))
        order_by_load = np.argsort(-cnt, kind="stable")
        tot = np.zeros(_NW, np.int64)
        fill = np.zeros(_NW, np.int64)
        spw = nstep // _NW
        assign = [[] for _ in range(_NW)]
        for t in order_by_load:
            cands = np.where(fill < spw)[0]
            w = cands[np.argmin(tot[cands])]
            assign[w].append(int(t))
            tot[w] += cnt[t]
            fill[w] += 1
        order = np.concatenate([np.asarray(a, np.int64) for a in assign])
    except jax.errors.TracerArrayConversionError:
        order = np.arange(nstep, dtype=np.int64)

    pad = jnp.zeros((nstep, 7), jnp.int32)
    row0 = (jnp.asarray(order, jnp.int32) * _SR)[:, None]
    sa = jnp.concatenate([sa_steps[order], row0, pad], axis=1).reshape(-1)

    i_step = steps(idx)
    ia8 = jnp.zeros((nstep, 2 * _SR, 8), jnp.int32).at[:, :, 0].set(i_step)
    ia8 = ia8[order]

    masks_flat = masks.reshape(-1)

    def mk(a):
        a3 = a.reshape(2, nstep, _SR)
        padz = jnp.zeros((nstep, 8), a.dtype)
        return jnp.concatenate([a3[0], a3[1], padz], axis=1).reshape(-1)

    return (masks_flat, ia8.reshape(-1), sa)


def _pick(vec, l):
    return vec[l]


def _sc_body(X, y, H1, L1, mflat, ia8, sa, Xo, yo,
             ia8_v, sa_v, m0_v, m1_v, yin_v, yout_v, slabA, slabB,
             b0, b1, b2, b3, b4, b5, b6, b7,
             sem_meta, sem_si, sem_so, sem_g):
    B, C, T = X.shape
    RPW = B // _NW
    NSL = RPW // _SR
    wid = lax.axis_index("s") * _NC + lax.axis_index("c")
    base = pl.multiple_of(wid * RPW, RPW)
    t0e = pl.multiple_of(wid * (NSL * 16), NSL * 16)
    t08 = pl.multiple_of(wid * (NSL * 64), NSL * 64)

    pltpu.async_copy(ia8.at[pl.ds(t08, NSL * 64)], ia8_v, sem_meta)
    pltpu.async_copy(sa.at[pl.ds(t0e, NSL * 16)], sa_v, sem_meta)
    pltpu.async_copy(mflat.at[pl.ds(base, RPW)], m0_v, sem_meta)
    pltpu.async_copy(mflat.at[pl.ds(B + base, RPW)], m1_v, sem_meta)
    pltpu.async_copy(y.at[pl.ds(base, RPW)], yin_v, sem_meta)
    pltpu.make_async_copy(sa.at[pl.ds(t0e, NSL * 16)], sa_v, sem_meta).wait()
    pltpu.make_async_copy(ia8.at[pl.ds(t08, NSL * 64)], ia8_v, sem_meta).wait()
    for ref in (m0_v, m1_v):
        pltpu.make_async_copy(mflat.at[pl.ds(base, RPW)], ref, sem_meta).wait()
    pltpu.make_async_copy(y.at[pl.ds(base, RPW)], yin_v, sem_meta).wait()

    for t in range(RPW // 16):
        sl = pl.ds(t * 16, 16)
        m0f = m0_v[sl].astype(jnp.float32)
        m1f = m1_v[sl].astype(jnp.float32)
        yout_v[sl] = yin_v[sl] - 2.0 * m0f - 4.0 * m1f
    pltpu.async_copy(yout_v, yo.at[pl.ds(base, RPW)], sem_meta)

    tabs = (H1, L1)
    banks = (b0, b1, b2, b3, b4, b5, b6, b7)

    def pick_step(s):
        moff = pl.multiple_of(s * 16, 16)
        sv = sa_v[pl.ds(moff, 16)]
        return tuple(_pick(sv, lanei) for lanei in range(2 * _SR + 1))

    def fire_gathers(s, sts):
        for lanei in range(2 * _SR):
            ch = lanei // _SR
            ioff = pl.multiple_of((s * (2 * _SR) + lanei) * 8, 8)

            @pl.when(sts[lanei] >= 0)
            def _():
                pltpu.async_copy(tabs[ch].at[ia8_v.at[pl.ds(ioff, 1)]],
                                 banks[lanei], sem_g)

    def fire_slab_in(sts, slab):
        pltpu.async_copy(X.at[pl.ds(sts[2 * _SR], _SR)], slab, sem_si)

    sts0 = pick_step(0)
    fire_slab_in(sts0, slabA)
    fire_gathers(0, sts0)

    def process_step(s, sts, slab, other):
        row0 = sts[2 * _SR]
        nxt = pick_step(jnp.minimum(s + 1, NSL - 1))

        @pl.when(s >= 1)
        def _():
            pltpu.make_async_copy(other, Xo.at[pl.ds(base, _SR)],
                                  sem_so).wait()

        @pl.when(s < NSL - 1)
        def _():
            fire_slab_in(nxt, other)

        pltpu.make_async_copy(X.at[pl.ds(base, _SR)], slab, sem_si).wait()
        for lanei in range(2 * _SR):
            ch = lanei // _SR

            @pl.when(sts[lanei] >= 0)
            def _():
                pltpu.make_async_copy(tabs[ch].at[ia8_v.at[pl.ds(0, 1)]],
                                      banks[lanei], sem_g).wait()

        for lanei in range(2 * _SR):
            ch = lanei // _SR
            l = lanei % _SR
            st_s = sts[lanei]

            @pl.when(st_s >= 0)
            def _():
                def rep(k, _):
                    off = k * 64
                    for u in range(4):
                        slab[l, ch, pl.ds(off + u * 16, 16)] = (
                            banks[lanei][0, pl.ds(st_s + off + u * 16, 16)])
                    return _

                lax.fori_loop(0, T // 64, rep, None)

        @pl.when(s < NSL - 1)
        def _():
            fire_gathers(s + 1, nxt)

        pltpu.async_copy(slab, Xo.at[pl.ds(row0, _SR)], sem_so)
        return nxt

    def step2(s2, sts):
        sts = process_step(2 * s2, sts, slabA, slabB)
        sts = process_step(2 * s2 + 1, sts, slabB, slabA)
        return sts

    lax.fori_loop(0, NSL // 2, step2, sts0)
    pltpu.make_async_copy(slabB, Xo.at[pl.ds(base, _SR)], sem_so).wait()
    pltpu.make_async_copy(yout_v, yo.at[pl.ds(base, RPW)], sem_meta).wait()


def kernel(X, y, H1, L1):
    B, C, T = X.shape
    try:
        with jax.ensure_compile_time_eval(), \
                jax.default_device(jax.local_devices(backend="cpu")[0]):
            meta = _glitch_meta(H1.shape[0], H1.shape[1], T, B)
        mflat, ia, sa = (jnp.asarray(np.asarray(a)) for a in meta)
    except Exception:
        mflat, ia, sa = _glitch_meta(H1.shape[0], H1.shape[1], T, B)
    RPW = B // _NW
    NSL = RPW // _SR
    mesh = plsc.VectorSubcoreMesh(core_axis_name="c", subcore_axis_name="s",
                                  num_cores=_NC, num_subcores=_NS)
    run = pl.kernel(
        _sc_body,
        out_type=(jax.ShapeDtypeStruct((B, C, T), jnp.float32),
                  jax.ShapeDtypeStruct((B,), jnp.float32)),
        mesh=mesh,
        scratch_types=[
            pltpu.VMEM((NSL * 64,), jnp.int32),
            pltpu.VMEM((NSL * 16,), jnp.int32),
            pltpu.VMEM((RPW,), jnp.int32),
            pltpu.VMEM((RPW,), jnp.int32),
            pltpu.VMEM((RPW,), jnp.float32),
            pltpu.VMEM((RPW,), jnp.float32),
            pltpu.VMEM((_SR, C, T), jnp.float32),
            pltpu.VMEM((_SR, C, T), jnp.float32),
            pltpu.VMEM((1, 4096), jnp.float32),
            pltpu.VMEM((1, 4096), jnp.float32),
            pltpu.VMEM((1, 4096), jnp.float32),
            pltpu.VMEM((1, 4096), jnp.float32),
            pltpu.VMEM((1, 4096), jnp.float32),
            pltpu.VMEM((1, 4096), jnp.float32),
            pltpu.VMEM((1, 4096), jnp.float32),
            pltpu.VMEM((1, 4096), jnp.float32),
            pltpu.SemaphoreType.DMA,
            pltpu.SemaphoreType.DMA,
            pltpu.SemaphoreType.DMA,
            pltpu.SemaphoreType.DMA,
        ],
        compiler_params=pltpu.CompilerParams(needs_layout_passes=False),
    )
    Xo, yo = run(X, y, H1, L1, mflat, ia, sa)
    return (Xo, yo)
